# baseline (speedup 1.0000x reference)
import functools

import jax
import jax.numpy as jnp
from jax import lax
from jax.experimental import pallas as pl
from jax.experimental.pallas import tpu as pltpu
from jax.experimental.pallas import tpu_sc as plsc

N = 10000
E = 320000
D = 128

NC = 2
NS = 16
NW = NC * NS
NPT = 313
NPAD = NW * NPT
CHUNK = 4000
NCHUNK = E // CHUNK
G = 128
MCAP = 4096
NEG = float("-inf")
EPS = 1e-5


def _segmax_body(y_hbm, src_hbm, dst_hbm, out_hbm,
                 schunk, dchunk, msrc, mdst, rows, acc, sem):
    wid = lax.axis_index("s") * NC + lax.axis_index("c")
    lo = wid * NPT

    def init_acc(i, _):
        acc[pl.ds(i * 16, 16)] = jnp.full((16,), NEG, jnp.float32)
        return 0
    lax.fori_loop(0, (NPT + 1) * D // 16, init_acc, 0)

    def init_msrc(i, _):
        msrc[pl.ds(i * 16, 16)] = jnp.zeros((16,), jnp.int32)
        return 0
    lax.fori_loop(0, MCAP // 16, init_msrc, 0)

    def chunk_body(c, _):
        pltpu.sync_copy(src_hbm.at[pl.ds(c * CHUNK, CHUNK)], schunk)
        pltpu.sync_copy(dst_hbm.at[pl.ds(c * CHUNK, CHUNK)], dchunk)

        def filt(i, cnt):
            dv = dchunk[pl.ds(i * 16, 16)]
            sv = schunk[pl.ds(i * 16, 16)]
            dl = dv - lo
            m = (dl >= 0) & (dl < NPT)
            pos = cnt + plsc.cumsum(m.astype(jnp.int32)) - 1
            plsc.store_scatter(msrc, [pos], sv, mask=m)
            plsc.store_scatter(mdst, [pos], dl, mask=m)
            return cnt + jnp.max(plsc.all_reduce_population_count(m))
        cnt = lax.fori_loop(0, CHUNK // 16, filt, jnp.int32(0))

        mdst[pl.ds(cnt, 16)] = jnp.full((16,), NPT, jnp.int32)

        ng = (cnt + (G - 1)) // G

        def gbody(g, _):
            pltpu.async_copy(y_hbm.at[msrc.at[pl.ds(g * G, G)]],
                             rows.at[pl.ds(0, G)], sem).wait()
            ec = jnp.minimum(G, cnt - g * G)

            def kbody(k, _):
                dvec = mdst[pl.ds(g * G + k * 16, 16)]
                for l in range(16):
                    base = dvec[l] * D
                    el = k * 16 + l
                    for j in range(D // 16):
                        a = acc[pl.ds(base + j * 16, 16)]
                        r = rows[el, pl.ds(j * 16, 16)]
                        acc[pl.ds(base + j * 16, 16)] = jnp.maximum(a, r)
                return 0
            lax.fori_loop(0, (ec + 15) // 16, kbody, 0)
            return 0
        lax.fori_loop(0, ng, gbody, 0)
        return 0
    lax.fori_loop(0, NCHUNK, chunk_body, 0)

    pltpu.sync_copy(acc.at[pl.ds(0, NPT * D)],
                    out_hbm.at[pl.ds(lo * D, NPT * D)])


@functools.lru_cache(maxsize=1)
def _make_sc_segmax():
    return functools.partial(
        pl.kernel,
        compiler_params=pltpu.CompilerParams(needs_layout_passes=False),
        mesh=plsc.VectorSubcoreMesh(core_axis_name="c", subcore_axis_name="s"),
        out_type=jax.ShapeDtypeStruct((NPAD * D,), jnp.float32),
        scratch_types=[
            pltpu.VMEM((CHUNK,), jnp.int32),
            pltpu.VMEM((CHUNK,), jnp.int32),
            pltpu.VMEM((MCAP,), jnp.int32),
            pltpu.VMEM((MCAP,), jnp.int32),
            pltpu.VMEM((G + 16, D), jnp.float32),
            pltpu.VMEM(((NPT + 1) * D,), jnp.float32),
            pltpu.SemaphoreType.DMA,
        ],
    )(_segmax_body)


def _sc_segmax(y, src, dst):
    return _make_sc_segmax()(y, src, dst)


def _bn(h, g, be):
    mu = jnp.mean(h, axis=0, keepdims=True)
    var = jnp.mean((h - mu) ** 2, axis=0, keepdims=True)
    return g * (h - mu) / jnp.sqrt(var + EPS) + be


def _tc_pre_body(x_ref, W1_ref, b1_ref, Wl_ref, bl_ref, gl_ref, bel_ref,
                 y1_ref, skip_ref):
    x = x_ref[...]
    xw = jnp.dot(x, Wl_ref[...], preferred_element_type=jnp.float32) \
        + bl_ref[...]
    skip_ref[...] = _bn(xw, gl_ref[...], bel_ref[...])
    y1_ref[...] = jnp.dot(x, W1_ref[...],
                          preferred_element_type=jnp.float32) + b1_ref[...]


def _tc_mid_body(agg_ref, g1_ref, be1_ref, W2_ref, b2_ref, y2_ref):
    h = agg_ref[...]
    h = jnp.where(h == NEG, 0.0, h)
    h = jnp.maximum(_bn(h, g1_ref[...], be1_ref[...]), 0.0)
    y2_ref[...] = jnp.dot(h, W2_ref[...],
                          preferred_element_type=jnp.float32) + b2_ref[...]


def _tc_post_body(agg_ref, skip_ref, g2_ref, be2_ref, out_ref):
    h = agg_ref[...]
    h = jnp.where(h == NEG, 0.0, h)
    h = _bn(h, g2_ref[...], be2_ref[...])
    out_ref[...] = jnp.maximum(h + skip_ref[...], 0.0)


_tc_pre = pl.pallas_call(
    _tc_pre_body,
    out_shape=[jax.ShapeDtypeStruct((N, D), jnp.float32),
               jax.ShapeDtypeStruct((N, D), jnp.float32)],
)

_tc_mid = pl.pallas_call(
    _tc_mid_body,
    out_shape=jax.ShapeDtypeStruct((N, D), jnp.float32),
)

_tc_post = pl.pallas_call(
    _tc_post_body,
    out_shape=jax.ShapeDtypeStruct((N, D), jnp.float32),
)


def kernel(x, edge_index, W1, b1, W2, b2, Wl, bl, g1, be1, g2, be2, gl, bel):
    src = edge_index[0]
    dst = edge_index[1]
    r = lambda v: v.reshape(1, D)
    y1, skip = _tc_pre(x, W1, r(b1), Wl, r(bl), r(gl), r(bel))
    agg1 = _sc_segmax(y1, src, dst).reshape(NPAD, D)[:N]
    y2 = _tc_mid(agg1, r(g1), r(be1), W2, r(b2))
    agg2 = _sc_segmax(y2, src, dst).reshape(NPAD, D)[:N]
    return _tc_post(agg2, skip, r(g2), r(be2))
